# baseline (speedup 1.0000x reference)
import functools
import math

import jax
import jax.numpy as jnp
from jax import lax
from jax.experimental import pallas as pl
from jax.experimental.pallas import tpu as pltpu
from jax.experimental.pallas import tpu_sc as plsc

D_MODEL = 64
SCALE = math.sqrt(D_MODEL)

NC = 2
NS = 16
NW = NC * NS
LANES = 16

N_TOK = 4096
N_POS = 200
RPW = N_TOK // NW
NB = 3
G = 2
MB = 2

_mesh = plsc.VectorSubcoreMesh(
    core_axis_name="c", subcore_axis_name="s", num_cores=NC, num_subcores=NS
)


@functools.partial(
    pl.kernel,
    out_type=jax.ShapeDtypeStruct((N_POS, D_MODEL, N_TOK), jnp.float32),
    mesh=_mesh,
    scratch_types=[
        pltpu.VMEM((N_POS, RPW), jnp.int32),
        pltpu.VMEM((NB, RPW, 2 * D_MODEL), jnp.float32),
        pltpu.VMEM((MB, D_MODEL, RPW + 1), jnp.float32),
        pltpu.SemaphoreType.DMA((NB,)),
        pltpu.SemaphoreType.DMA((MB,)),
    ],
    compiler_params=pltpu.CompilerParams(needs_layout_passes=False),
)
def _emb_lookup(xt_hbm, tw_hbm, out_hbm, idx_v, rows_v, obuf_v, gsem, ssem):
    wid = lax.axis_index("s") * NC + lax.axis_index("c")
    base = wid * RPW
    pltpu.sync_copy(xt_hbm.at[:, pl.ds(base, RPW)], idx_v)

    lane = lax.iota(jnp.int32, LANES)
    rows16 = [lane + (k * LANES) for k in range(RPW // LANES)]

    def gather(p, b):
        return pltpu.make_async_copy(
            tw_hbm.at[idx_v.at[p]], rows_v.at[b], gsem.at[b]
        )

    def store(p, m):
        return pltpu.make_async_copy(
            obuf_v.at[m, :, pl.ds(0, RPW)],
            out_hbm.at[p, :, pl.ds(base, RPW)],
            ssem.at[m],
        )

    for p in range(G):
        gather(p, p % NB).start()

    def chunk_body(p, carry):
        b = p % NB
        m = p % MB

        @pl.when(p + G < N_POS)
        def _launch():
            gather(p + G, (p + G) % NB).start()

        gather(p, b).wait()

        @pl.when(p >= MB)
        def _drain():
            store(p - MB, m).wait()

        m16 = lax.broadcast_in_dim(m, (LANES,), ())

        @plsc.parallel_loop(0, RPW, unroll=4)
        def _tr(j):
            j16 = lax.broadcast_in_dim(j, (LANES,), ())
            for k in range(D_MODEL // LANES):
                vals = rows_v[b, j, pl.ds(k * LANES, LANES)]
                plsc.store_scatter(obuf_v, [m16, rows16[k], j16], vals)

        store(p, m).start()
        return carry

    lax.fori_loop(0, N_POS, chunk_body, 0)

    for p in range(N_POS - MB, N_POS):
        store(p, p % MB).wait()


def kernel(x, table):
    xt = jnp.transpose(x.astype(jnp.int32))
    tw = jnp.concatenate([table, table], axis=1) * SCALE
    out = _emb_lookup(xt, tw)
    return jnp.transpose(out, (2, 0, 1))


# speedup vs baseline: 1.1329x; 1.1329x over previous
import functools
import math

import jax
import jax.numpy as jnp
from jax import lax
from jax.experimental import pallas as pl
from jax.experimental.pallas import tpu as pltpu
from jax.experimental.pallas import tpu_sc as plsc

D_MODEL = 64
SCALE = math.sqrt(D_MODEL)

NC = 2
NS = 16
NW = NC * NS
LANES = 16

B_TOTAL = 4096 * 200
BPW = B_TOTAL // NW
CHUNK = 128
NCHUNK = BPW // CHUNK
NB = 6
G = 4

_mesh = plsc.VectorSubcoreMesh(
    core_axis_name="c", subcore_axis_name="s", num_cores=NC, num_subcores=NS
)


@functools.partial(
    pl.kernel,
    out_type=jax.ShapeDtypeStruct((B_TOTAL, D_MODEL), jnp.float32),
    mesh=_mesh,
    scratch_types=[
        pltpu.VMEM((NCHUNK, CHUNK), jnp.int32),
        pltpu.VMEM((NB, CHUNK, D_MODEL), jnp.float32),
        pltpu.SemaphoreType.DMA((NB,)),
        pltpu.SemaphoreType.DMA((NB,)),
    ],
    compiler_params=pltpu.CompilerParams(use_tc_tiling_on_sc=False),
)
def _emb_lookup(x_hbm, table_hbm, out_hbm, idx_v, rows_v, gsem, ssem):
    wid = lax.axis_index("s") * NC + lax.axis_index("c")
    base = wid * BPW
    pltpu.sync_copy(x_hbm.at[wid], idx_v)

    def gather(c, b):
        return pltpu.make_async_copy(
            table_hbm.at[idx_v.at[c]], rows_v.at[b], gsem.at[b]
        )

    def store(c, b):
        return pltpu.make_async_copy(
            rows_v.at[b], out_hbm.at[pl.ds(base + c * CHUNK, CHUNK)], ssem.at[b]
        )

    for c in range(G):
        gather(c, c % NB).start()

    def chunk_body(c, carry):
        b = c % NB
        cg = c + G
        bg = cg % NB

        @pl.when(cg < NCHUNK)
        def _launch():
            @pl.when(cg >= NB)
            def _drain():
                store(cg - NB, bg).wait()

            gather(cg, bg).start()

        gather(c, b).wait()

        @plsc.parallel_loop(0, CHUNK, unroll=8)
        def _scale(i):
            for j in range(D_MODEL // LANES):
                sl = pl.ds(j * LANES, LANES)
                rows_v[b, i, sl] = rows_v[b, i, sl] * SCALE
        store(c, b).start()
        return carry

    lax.fori_loop(0, NCHUNK, chunk_body, 0)

    for k in range(NB):
        c = NCHUNK - NB + k
        store(c, c % NB).wait()


def kernel(x, table):
    x_flat = x.reshape(NW, NCHUNK, CHUNK).astype(jnp.int32)
    out = _emb_lookup(x_flat, table)
    return out.reshape(x.shape + (D_MODEL,))
